# Initial kernel scaffold; baseline (speedup 1.0000x reference)
import functools

import jax
import jax.numpy as jnp
from jax import lax
from jax.experimental import pallas as pl
from jax.experimental.pallas import tpu as pltpu
from jax.experimental.pallas import tpu_sc as plsc

_NUM_EXPERTS = 3
_B = 32768
_F = 16
_LANES = 16
_NC = 2
_NS = 16
_NW = _NC * _NS
_ROWS = _B // _NW
_GROUPS = _ROWS // _LANES

_mesh = plsc.VectorSubcoreMesh(core_axis_name="c", subcore_axis_name="s")


@functools.partial(
    pl.kernel,
    mesh=_mesh,
    out_type=jax.ShapeDtypeStruct((_B, _NUM_EXPERTS), jnp.float32),
    scratch_types=[
        pltpu.VMEM((_ROWS, _F), jnp.float32),
        pltpu.VMEM((_ROWS, _NUM_EXPERTS), jnp.float32),
    ],
)
def _gating_kernel(feat_hbm, out_hbm, feat_v, out_v):
    wid = lax.axis_index("s") * _NC + lax.axis_index("c")
    base = wid * _ROWS
    pltpu.sync_copy(feat_hbm.at[pl.ds(base, _ROWS), :], feat_v)

    lanes = lax.iota(jnp.int32, _LANES)
    col4 = jnp.full((_LANES,), 4, jnp.int32)
    col5 = jnp.full((_LANES,), 5, jnp.int32)
    col6 = jnp.full((_LANES,), 6, jnp.int32)
    col7 = jnp.full((_LANES,), 7, jnp.int32)
    col11 = jnp.full((_LANES,), 11, jnp.int32)
    out0 = jnp.full((_LANES,), 0, jnp.int32)
    out1 = jnp.full((_LANES,), 1, jnp.int32)
    out2 = jnp.full((_LANES,), 2, jnp.int32)

    def body(g, carry):
        rows = lanes + g * _LANES
        t = plsc.load_gather(feat_v, [rows, col11])
        a = plsc.load_gather(feat_v, [rows, col4])
        b = plsc.load_gather(feat_v, [rows, col5])
        c = plsc.load_gather(feat_v, [rows, col6])
        d = plsc.load_gather(feat_v, [rows, col7])
        trend = t > 0.5
        cyc = (a + b) > (c + d)
        w0 = jnp.where(trend, 1.0, 0.0).astype(jnp.float32)
        w1 = jnp.where(jnp.logical_and(jnp.logical_not(trend), cyc),
                       1.0, 0.0).astype(jnp.float32)
        w2 = 1.0 - w0 - w1
        plsc.store_scatter(out_v, [rows, out0], w0)
        plsc.store_scatter(out_v, [rows, out1], w1)
        plsc.store_scatter(out_v, [rows, out2], w2)
        return carry

    lax.fori_loop(0, _GROUPS, body, 0)
    pltpu.sync_copy(out_v, out_hbm.at[pl.ds(base, _ROWS), :])


def kernel(features):
    return _gating_kernel(features)


# speedup vs baseline: 1.8362x; 1.8362x over previous
import functools

import jax
import jax.numpy as jnp
from jax import lax
from jax.experimental import pallas as pl
from jax.experimental.pallas import tpu as pltpu
from jax.experimental.pallas import tpu_sc as plsc

_NUM_EXPERTS = 3
_B = 32768
_F = 16
_LANES = 16
_NC = 2
_NS = 16
_NW = _NC * _NS
_ROWS = _B // _NW
_GROUPS = _ROWS // _LANES

_mesh = plsc.VectorSubcoreMesh(core_axis_name="c", subcore_axis_name="s")


@functools.partial(
    pl.kernel,
    mesh=_mesh,
    compiler_params=pltpu.CompilerParams(needs_layout_passes=False),
    out_type=jax.ShapeDtypeStruct((_B * _NUM_EXPERTS,), jnp.float32),
    scratch_types=[
        pltpu.VMEM((_ROWS * _F,), jnp.float32),
        pltpu.VMEM((_ROWS * _NUM_EXPERTS,), jnp.float32),
    ],
)
def _gating_kernel(feat_hbm, out_hbm, feat_v, out_v):
    wid = lax.axis_index("s") * _NC + lax.axis_index("c")
    pltpu.sync_copy(feat_hbm.at[pl.ds(wid * (_ROWS * _F), _ROWS * _F)], feat_v)

    lanes = lax.iota(jnp.int32, _LANES)

    def body(g, carry):
        fbase = (lanes + g * _LANES) * _F
        t = plsc.load_gather(feat_v, [fbase + 11])
        a = plsc.load_gather(feat_v, [fbase + 4])
        b = plsc.load_gather(feat_v, [fbase + 5])
        c = plsc.load_gather(feat_v, [fbase + 6])
        d = plsc.load_gather(feat_v, [fbase + 7])
        trend = t > 0.5
        cyc = (a + b) > (c + d)
        w0 = jnp.where(trend, 1.0, 0.0).astype(jnp.float32)
        w1 = jnp.where(jnp.logical_and(jnp.logical_not(trend), cyc),
                       1.0, 0.0).astype(jnp.float32)
        w2 = 1.0 - w0 - w1
        obase = (lanes + g * _LANES) * _NUM_EXPERTS
        plsc.store_scatter(out_v, [obase], w0)
        plsc.store_scatter(out_v, [obase + 1], w1)
        plsc.store_scatter(out_v, [obase + 2], w2)
        return carry

    lax.fori_loop(0, _GROUPS, body, 0)
    pltpu.sync_copy(
        out_v, out_hbm.at[pl.ds(wid * (_ROWS * _NUM_EXPERTS), _ROWS * _NUM_EXPERTS)])


def kernel(features):
    flat = features.reshape(_B * _F)
    return _gating_kernel(flat).reshape(_B, _NUM_EXPERTS)


# speedup vs baseline: 5.9615x; 3.2467x over previous
import functools

import jax
import jax.numpy as jnp
from jax import lax
from jax.experimental import pallas as pl
from jax.experimental.pallas import tpu as pltpu
from jax.experimental.pallas import tpu_sc as plsc

_NUM_EXPERTS = 3
_B = 32768
_F = 16
_LANES = 16
_NC = 2
_NS = 16
_NW = _NC * _NS
_ROWS = _B // _NW
_GROUPS = _ROWS // _LANES

_mesh = plsc.VectorSubcoreMesh(core_axis_name="c", subcore_axis_name="s")


@functools.partial(
    pl.kernel,
    mesh=_mesh,
    compiler_params=pltpu.CompilerParams(needs_layout_passes=False),
    out_type=jax.ShapeDtypeStruct((_NUM_EXPERTS, _B), jnp.float32),
    scratch_types=[
        pltpu.VMEM((_F, _ROWS), jnp.float32),
        pltpu.VMEM((_NUM_EXPERTS, _ROWS), jnp.float32),
    ],
)
def _gating_kernel(xt_hbm, out_hbm, feat_v, out_v):
    wid = lax.axis_index("s") * _NC + lax.axis_index("c")
    base = wid * _ROWS
    pltpu.sync_copy(xt_hbm.at[:, pl.ds(base, _ROWS)], feat_v)

    def body(g, carry):
        sl = pl.ds(g * _LANES, _LANES)
        t = feat_v[11, sl]
        a = feat_v[4, sl]
        b = feat_v[5, sl]
        c = feat_v[6, sl]
        d = feat_v[7, sl]
        trend = t > 0.5
        cyc = (a + b) > (c + d)
        w0 = jnp.where(trend, 1.0, 0.0).astype(jnp.float32)
        w1 = jnp.where(jnp.logical_and(jnp.logical_not(trend), cyc),
                       1.0, 0.0).astype(jnp.float32)
        w2 = 1.0 - w0 - w1
        out_v[0, sl] = w0
        out_v[1, sl] = w1
        out_v[2, sl] = w2
        return carry

    lax.fori_loop(0, _GROUPS, body, 0)
    pltpu.sync_copy(out_v, out_hbm.at[:, pl.ds(base, _ROWS)])


def kernel(features):
    return _gating_kernel(features.T).T


# speedup vs baseline: 6.0044x; 1.0072x over previous
import functools

import jax
import jax.numpy as jnp
from jax import lax
from jax.experimental import pallas as pl
from jax.experimental.pallas import tpu as pltpu
from jax.experimental.pallas import tpu_sc as plsc

_NUM_EXPERTS = 3
_B = 32768
_F = 16
_LANES = 16
_NC = 2
_NS = 16
_NW = _NC * _NS
_ROWS = _B // _NW
_GROUPS = _ROWS // _LANES

_mesh = plsc.VectorSubcoreMesh(core_axis_name="c", subcore_axis_name="s")


@functools.partial(
    pl.kernel,
    mesh=_mesh,
    compiler_params=pltpu.CompilerParams(needs_layout_passes=False),
    out_type=jax.ShapeDtypeStruct((_NUM_EXPERTS, _B), jnp.float32),
    scratch_types=[
        pltpu.VMEM((4, _ROWS), jnp.float32),
        pltpu.VMEM((1, _ROWS), jnp.float32),
        pltpu.VMEM((_NUM_EXPERTS, _ROWS), jnp.float32),
        pltpu.SemaphoreType.DMA,
        pltpu.SemaphoreType.DMA,
    ],
)
def _gating_kernel(xt_hbm, out_hbm, v47, v11, out_v, sem_a, sem_b):
    wid = lax.axis_index("s") * _NC + lax.axis_index("c")
    base = wid * _ROWS
    cp_a = pltpu.async_copy(xt_hbm.at[pl.ds(4, 4), pl.ds(base, _ROWS)], v47, sem_a)
    cp_b = pltpu.async_copy(xt_hbm.at[pl.ds(11, 1), pl.ds(base, _ROWS)], v11, sem_b)
    cp_a.wait()
    cp_b.wait()

    for g in range(_GROUPS):
        sl = pl.ds(g * _LANES, _LANES)
        t = v11[0, sl]
        a = v47[0, sl]
        b = v47[1, sl]
        c = v47[2, sl]
        d = v47[3, sl]
        trend = t > 0.5
        cyc = (a + b) > (c + d)
        w0 = jnp.where(trend, 1.0, 0.0).astype(jnp.float32)
        w1 = jnp.where(jnp.logical_and(jnp.logical_not(trend), cyc),
                       1.0, 0.0).astype(jnp.float32)
        w2 = 1.0 - w0 - w1
        out_v[0, sl] = w0
        out_v[1, sl] = w1
        out_v[2, sl] = w2

    pltpu.sync_copy(out_v, out_hbm.at[:, pl.ds(base, _ROWS)])


def kernel(features):
    return _gating_kernel(features.T).T


# speedup vs baseline: 6.0762x; 1.0120x over previous
import functools

import jax
import jax.numpy as jnp
from jax import lax
from jax.experimental import pallas as pl
from jax.experimental.pallas import tpu as pltpu
from jax.experimental.pallas import tpu_sc as plsc

_NUM_EXPERTS = 3
_B = 32768
_F = 16
_LANES = 16
_NC = 2
_NS = 16
_NW = _NC * _NS
_ROWS = _B // _NW
_GROUPS = _ROWS // _LANES

_mesh = plsc.VectorSubcoreMesh(core_axis_name="c", subcore_axis_name="s")


@functools.partial(
    pl.kernel,
    mesh=_mesh,
    compiler_params=pltpu.CompilerParams(needs_layout_passes=False),
    out_type=jax.ShapeDtypeStruct((_NUM_EXPERTS, _B), jnp.float32),
    scratch_types=[
        pltpu.VMEM((4, _ROWS), jnp.float32),
        pltpu.VMEM((1, _ROWS), jnp.float32),
        pltpu.VMEM((_NUM_EXPERTS, _ROWS), jnp.float32),
        pltpu.SemaphoreType.DMA,
        pltpu.SemaphoreType.DMA,
        pltpu.SemaphoreType.DMA,
        pltpu.SemaphoreType.DMA,
        pltpu.SemaphoreType.DMA,
    ],
)
def _gating_kernel(xt_hbm, out_hbm, v47, v11, out_v,
                   sem_a0, sem_b0, sem_a1, sem_b1, sem_o):
    wid = lax.axis_index("s") * _NC + lax.axis_index("c")
    base = wid * _ROWS
    half = _ROWS // 2

    in_sems = ((sem_a0, sem_b0), (sem_a1, sem_b1))
    cps = []
    for h in range(2):
        cps.append((
            pltpu.async_copy(
                xt_hbm.at[pl.ds(4, 4), pl.ds(base + h * half, half)],
                v47.at[:, pl.ds(h * half, half)], in_sems[h][0]),
            pltpu.async_copy(
                xt_hbm.at[pl.ds(11, 1), pl.ds(base + h * half, half)],
                v11.at[:, pl.ds(h * half, half)], in_sems[h][1]),
        ))

    out_cps = []
    for h in range(2):
        cps[h][0].wait()
        cps[h][1].wait()
        for g in range(h * _GROUPS // 2, (h + 1) * _GROUPS // 2):
            sl = pl.ds(g * _LANES, _LANES)
            t = v11[0, sl]
            a = v47[0, sl]
            b = v47[1, sl]
            c = v47[2, sl]
            d = v47[3, sl]
            trend = t > 0.5
            cyc = (a + b) > (c + d)
            w0 = jnp.where(trend, 1.0, 0.0).astype(jnp.float32)
            w1 = jnp.where(jnp.logical_and(jnp.logical_not(trend), cyc),
                           1.0, 0.0).astype(jnp.float32)
            w2 = 1.0 - w0 - w1
            out_v[0, sl] = w0
            out_v[1, sl] = w1
            out_v[2, sl] = w2
        out_cps.append(pltpu.async_copy(
            out_v.at[:, pl.ds(h * half, half)],
            out_hbm.at[:, pl.ds(base + h * half, half)], sem_o))

    for cp in out_cps:
        cp.wait()


def kernel(features):
    return _gating_kernel(features.T).T
